# Initial kernel scaffold; baseline (speedup 1.0000x reference)
import jax
import jax.numpy as jnp
from jax.experimental import pallas as pl


def kernel(x, Q, a_val, ia_val, params, a_src, a_dst, ia_src, ia_dst, r_src, r_dst, c_src, c_dst):
    raise NotImplementedError("write your pallas kernel here")


# speedup vs baseline: 90.6879x; 90.6879x over previous
import jax
import jax.numpy as jnp
from jax.experimental import pallas as pl

H_IMG = 128
W_IMG = 128
N = H_IMG * W_IMG
C_IN = 200
HIDE = 128
S_GRID = 32
S = S_GRID * S_GRID
NCLS = 16
DOUT = HIDE // 2

TILES = 16
RPT = H_IMG // TILES
PPT = N // TILES
SPT = S // TILES

_RS = float(1.0 / jnp.sqrt(1.0 + 1e-05))
_NEG = -1e30


def _lrelu(x):
    return jnp.where(x >= 0, x, 0.01 * x)


def _ln(x):
    m = jnp.mean(x, axis=-1, keepdims=True)
    v = jnp.mean((x - m) ** 2, axis=-1, keepdims=True)
    return (x - m) / jnp.sqrt(v + 1e-05)


def _pool_matrix():
    s_idx = jax.lax.broadcasted_iota(jnp.int32, (SPT, PPT), 0)
    p_idx = jax.lax.broadcasted_iota(jnp.int32, (SPT, PPT), 1)
    sp = (p_idx // (W_IMG * 4)) * S_GRID + (p_idx % W_IMG) // 4
    return jnp.where(sp == s_idx, 1.0 / 16.0, 0.0).astype(jnp.float32)


def _bcast_matrix():
    p_idx = jax.lax.broadcasted_iota(jnp.int32, (PPT, SPT), 0)
    s_idx = jax.lax.broadcasted_iota(jnp.int32, (PPT, SPT), 1)
    sp = (p_idx // (W_IMG * 4)) * S_GRID + (p_idx % W_IMG) // 4
    return jnp.where(sp == s_idx, 1.0, 0.0).astype(jnp.float32)


def _pre_kernel(x_ref, w_ref, b_ref, g_ref, bb_ref, h_ref, hp_ref):
    x = x_ref[...]
    h = jnp.dot(x, w_ref[...], preferred_element_type=jnp.float32) + b_ref[...]
    h = h * (g_ref[...] * _RS) + bb_ref[...]
    h_ref[...] = h
    hp_ref[...] = jnp.dot(_pool_matrix(), h, preferred_element_type=jnp.float32)


def _gnn_kernel(hp_ref, a1_ref, a2_ref, w_ref, b_ref, g_ref, be_ref, out_ref):
    hp = hp_ref[...]
    a1 = a1_ref[...]
    a2 = a2_ref[...]
    for i in range(5):
        hl = jnp.dot(hp, w_ref[i], preferred_element_type=jnp.float32) + b_ref[i : i + 1, :]
        o = jnp.dot(a1, hl, preferred_element_type=jnp.float32)
        o = o + jnp.dot(a2, hp, preferred_element_type=jnp.float32)
        o = o * (_RS * g_ref[i : i + 1, :]) + be_ref[i : i + 1, :]
        hp = _lrelu(o)
    out_ref[...] = hp


def _projections(z, wrv_ref, brv_ref, wcv_ref, bcv_ref, wrq_ref, brq_ref,
                 wcq_ref, bcq_ref, rowqv_ref, colqv_ref):
    rv = jnp.dot(z, wrv_ref[...], preferred_element_type=jnp.float32) + brv_ref[...]
    cv = jnp.dot(z, wcv_ref[...], preferred_element_type=jnp.float32) + bcv_ref[...]
    rq = _ln(jnp.dot(z, wrq_ref[...], preferred_element_type=jnp.float32) + brq_ref[...])
    cq = _ln(jnp.dot(z, wcq_ref[...], preferred_element_type=jnp.float32) + bcq_ref[...])
    rowqv_ref[...] = jnp.concatenate([rq, rv], axis=-1).reshape(RPT, W_IMG, 2 * DOUT)
    colqv_ref[...] = jnp.concatenate([cq, cv], axis=-1).reshape(RPT, W_IMG, 2 * DOUT)


def _proj_kernel(z_ref, wrv_ref, brv_ref, wcv_ref, bcv_ref, wrq_ref, brq_ref,
                 wcq_ref, bcq_ref, rowqv_ref, colqv_ref):
    _projections(z_ref[...], wrv_ref, brv_ref, wcv_ref, bcv_ref, wrq_ref,
                 brq_ref, wcq_ref, bcq_ref, rowqv_ref, colqv_ref)


def _combine(ro_ref, co_ref, pg_ref, pb_ref):
    ro = ro_ref[...].reshape(PPT, DOUT)
    co = co_ref[...].reshape(PPT, DOUT)
    zc = jnp.concatenate([ro, co], axis=-1)
    return _lrelu(zc * (_RS * pg_ref[...]) + pb_ref[...])


def _proj_combine_kernel(ro_ref, co_ref, pg_ref, pb_ref, wrv_ref, brv_ref,
                         wcv_ref, bcv_ref, wrq_ref, brq_ref, wcq_ref, bcq_ref,
                         rowqv_ref, colqv_ref):
    z = _combine(ro_ref, co_ref, pg_ref, pb_ref)
    _projections(z, wrv_ref, brv_ref, wcv_ref, bcv_ref, wrq_ref, brq_ref,
                 wcq_ref, bcq_ref, rowqv_ref, colqv_ref)


def _band_attention(q, v):
    i = jax.lax.broadcasted_iota(jnp.int32, (W_IMG, W_IMG), 0)
    j = jax.lax.broadcasted_iota(jnp.int32, (W_IMG, W_IMG), 1)
    band = jnp.abs(i - j) <= 8
    s = jax.lax.dot_general(q, q, (((1,), (1,)), ((), ())),
                            preferred_element_type=jnp.float32) * (1.0 / DOUT)
    s = jnp.where(band, s, _NEG)
    m = jnp.max(s, axis=1, keepdims=True)
    e = jnp.exp(s - m)
    den = jnp.sum(e, axis=1, keepdims=True) + 1e-16
    p = e / den
    return jnp.dot(p, v, preferred_element_type=jnp.float32)


def _row_attn_kernel(qv_ref, out_ref):
    for i in range(RPT):
        q = qv_ref[i, :, :DOUT]
        v = qv_ref[i, :, DOUT:]
        out_ref[i, :, :] = _band_attention(q, v)


def _col_attn_kernel(qv_ref, out_ref):
    for i in range(RPT):
        q = qv_ref[:, i, :DOUT]
        v = qv_ref[:, i, DOUT:]
        out_ref[:, i, :] = _band_attention(q, v)


def _final_kernel(ro_ref, co_ref, pg_ref, pb_ref, hp_ref, wc_ref, bc_ref, out_ref):
    z = _combine(ro_ref, co_ref, pg_ref, pb_ref)
    hyp = jnp.dot(_bcast_matrix(), hp_ref[...], preferred_element_type=jnp.float32)
    h1 = hyp + z
    logits = jnp.dot(h1, wc_ref[...], preferred_element_type=jnp.float32) + bc_ref[...]
    m = jnp.max(logits, axis=-1, keepdims=True)
    e = jnp.exp(logits - m)
    out_ref[...] = e / jnp.sum(e, axis=-1, keepdims=True)


def _full(shape):
    nd = len(shape)
    return pl.BlockSpec(shape, lambda *k, _nd=nd: (0,) * _nd)


def _attention_round(rowqv, colqv):
    f32 = jnp.float32
    rowout = pl.pallas_call(
        _row_attn_kernel,
        grid=(TILES,),
        in_specs=[pl.BlockSpec((RPT, W_IMG, 2 * DOUT), lambda k: (k, 0, 0))],
        out_specs=pl.BlockSpec((RPT, W_IMG, DOUT), lambda k: (k, 0, 0)),
        out_shape=jax.ShapeDtypeStruct((H_IMG, W_IMG, DOUT), f32),
    )(rowqv)
    colout = pl.pallas_call(
        _col_attn_kernel,
        grid=(TILES,),
        in_specs=[pl.BlockSpec((H_IMG, RPT, 2 * DOUT), lambda k: (0, k, 0))],
        out_specs=pl.BlockSpec((H_IMG, RPT, DOUT), lambda k: (0, k, 0)),
        out_shape=jax.ShapeDtypeStruct((H_IMG, W_IMG, DOUT), f32),
    )(colqv)
    return rowout, colout


def kernel(x, Q, a_val, ia_val, params, a_src, a_dst, ia_src, ia_dst,
           r_src, r_dst, c_src, c_dst):
    p = params
    f32 = jnp.float32

    a1 = jnp.zeros((S, S), f32).at[a_dst, a_src].add(a_val)
    a2 = jnp.zeros((S, S), f32).at[ia_dst, ia_src].add(ia_val)

    row2 = lambda a: a.reshape(1, -1)

    h, hp = pl.pallas_call(
        _pre_kernel,
        grid=(TILES,),
        in_specs=[
            pl.BlockSpec((PPT, C_IN), lambda k: (k, 0)),
            _full((C_IN, HIDE)),
            _full((1, HIDE)),
            _full((1, HIDE)),
            _full((1, HIDE)),
        ],
        out_specs=[
            pl.BlockSpec((PPT, HIDE), lambda k: (k, 0)),
            pl.BlockSpec((SPT, HIDE), lambda k: (k, 0)),
        ],
        out_shape=[
            jax.ShapeDtypeStruct((N, HIDE), f32),
            jax.ShapeDtypeStruct((S, HIDE), f32),
        ],
    )(x, p['pre_W'], row2(p['pre_b']), row2(p['bn0_g']), row2(p['bn0_b']))

    hp = pl.pallas_call(
        _gnn_kernel,
        in_specs=[
            _full((S, HIDE)),
            _full((S, S)),
            _full((S, S)),
            _full((5, HIDE, HIDE)),
            _full((5, HIDE)),
            _full((5, HIDE)),
            _full((5, HIDE)),
        ],
        out_specs=_full((S, HIDE)),
        out_shape=jax.ShapeDtypeStruct((S, HIDE), f32),
    )(hp, a1, a2, p['mm_W'], p['mm_b'], p['mm_g'], p['mm_be'])

    qv_shapes = [
        jax.ShapeDtypeStruct((H_IMG, W_IMG, 2 * DOUT), f32),
        jax.ShapeDtypeStruct((H_IMG, W_IMG, 2 * DOUT), f32),
    ]
    qv_specs = [
        pl.BlockSpec((RPT, W_IMG, 2 * DOUT), lambda k: (k, 0, 0)),
        pl.BlockSpec((RPT, W_IMG, 2 * DOUT), lambda k: (k, 0, 0)),
    ]
    wspecs = [_full((HIDE, DOUT)), _full((1, DOUT))] * 4
    weights0 = [p['psf_Wrv'][0], row2(p['psf_brv'][0]),
                p['psf_Wcv'][0], row2(p['psf_bcv'][0]),
                p['psf_Wrq'][0], row2(p['psf_brq'][0]),
                p['psf_Wcq'][0], row2(p['psf_bcq'][0])]
    rowqv, colqv = pl.pallas_call(
        _proj_kernel,
        grid=(TILES,),
        in_specs=[pl.BlockSpec((PPT, HIDE), lambda k: (k, 0))] + wspecs,
        out_specs=qv_specs,
        out_shape=qv_shapes,
    )(h, *weights0)

    rowout, colout = _attention_round(rowqv, colqv)

    weights1 = [p['psf_Wrv'][1], row2(p['psf_brv'][1]),
                p['psf_Wcv'][1], row2(p['psf_bcv'][1]),
                p['psf_Wrq'][1], row2(p['psf_brq'][1]),
                p['psf_Wcq'][1], row2(p['psf_bcq'][1])]
    out3_specs = [
        pl.BlockSpec((RPT, W_IMG, DOUT), lambda k: (k, 0, 0)),
        pl.BlockSpec((RPT, W_IMG, DOUT), lambda k: (k, 0, 0)),
    ]
    rowqv, colqv = pl.pallas_call(
        _proj_combine_kernel,
        grid=(TILES,),
        in_specs=out3_specs + [_full((1, HIDE)), _full((1, HIDE))] + wspecs,
        out_specs=qv_specs,
        out_shape=qv_shapes,
    )(rowout, colout, row2(p['psf_g'][0]), row2(p['psf_b2'][0]), *weights1)

    rowout, colout = _attention_round(rowqv, colqv)

    out = pl.pallas_call(
        _final_kernel,
        grid=(TILES,),
        in_specs=out3_specs + [
            _full((1, HIDE)),
            _full((1, HIDE)),
            pl.BlockSpec((SPT, HIDE), lambda k: (k, 0)),
            _full((HIDE, NCLS)),
            _full((1, NCLS)),
        ],
        out_specs=pl.BlockSpec((PPT, NCLS), lambda k: (k, 0)),
        out_shape=jax.ShapeDtypeStruct((N, NCLS), f32),
    )(rowout, colout, row2(p['psf_g'][1]), row2(p['psf_b2'][1]), hp,
      p['cls_W'], row2(p['cls_b']))

    return out
